# Initial kernel scaffold; baseline (speedup 1.0000x reference)
import jax
import jax.numpy as jnp
from jax.experimental import pallas as pl


def kernel(x, Wg, bg, Wn, bn, A, B):
    raise NotImplementedError("write your pallas kernel here")


# speedup vs baseline: 3.0933x; 3.0933x over previous
import functools

import jax
import jax.numpy as jnp
from jax.experimental import pallas as pl

_NE = 8
_R = 8
_SCALING = 1.0
_PAD = 128

_SORT_NET = (
    (0, 1), (2, 3), (4, 5), (6, 7),
    (0, 2), (1, 3), (4, 6), (5, 7),
    (1, 2), (5, 6),
    (0, 4), (1, 5), (2, 6), (3, 7),
    (2, 4), (3, 5),
    (1, 2), (3, 4), (5, 6),
)


def _moe_body(x_ref, p_ref, q_ref, noise_ref, bg_ref, bn_ref, o_ref):
    t = x_ref.shape[0]
    ner = _NE * _R
    g = jnp.dot(x_ref[...], p_ref[...], preferred_element_type=jnp.float32)

    logits = g[:, ner:ner + _NE] + bg_ref[0, :]
    nlogits = g[:, ner + _NE:ner + 2 * _NE] + bn_ref[0, :]
    softplus = jnp.maximum(nlogits, 0.0) + jnp.log1p(jnp.exp(-jnp.abs(nlogits)))
    noisy = logits + noise_ref[...] * softplus

    m = jnp.max(noisy, axis=1, keepdims=True)
    e = jnp.exp(noisy - m)
    p = e / jnp.sum(e, axis=1, keepdims=True)

    cols = [p[:, i:i + 1] for i in range(_NE)]
    for a, b in _SORT_NET:
        hi = jnp.maximum(cols[a], cols[b])
        lo = jnp.minimum(cols[a], cols[b])
        cols[a], cols[b] = hi, lo

    scale = jnp.concatenate(
        [jnp.broadcast_to(c, (t, _R)) for c in cols]
        + [jnp.zeros((t, _PAD - ner), jnp.float32)],
        axis=1,
    )
    o_ref[...] = jnp.dot(g * scale, q_ref[...],
                         preferred_element_type=jnp.float32)


@functools.partial(jax.jit, static_argnames=())
def kernel(x, Wg, bg, Wn, bn, A, B):
    n_tokens, n_embed = x.shape
    ner = _NE * _R

    a_all = A.reshape(ner, n_embed)
    proj = jnp.concatenate([a_all, Wg, Wn], axis=0)
    proj = jnp.pad(proj, ((0, _PAD - ner - 2 * _NE), (0, 0))).T

    b_all = (B.transpose(0, 2, 1).reshape(ner, n_embed)) * _SCALING
    b_all = jnp.pad(b_all, ((0, _PAD - ner), (0, 0)))

    noise = jax.random.normal(jax.random.key(42), (n_tokens, _NE), jnp.float32)

    tile = 1024
    grid = (n_tokens // tile,)
    return pl.pallas_call(
        _moe_body,
        grid=grid,
        in_specs=[
            pl.BlockSpec((tile, n_embed), lambda i: (i, 0)),
            pl.BlockSpec((n_embed, _PAD), lambda i: (0, 0)),
            pl.BlockSpec((_PAD, n_embed), lambda i: (0, 0)),
            pl.BlockSpec((tile, _NE), lambda i: (i, 0)),
            pl.BlockSpec((1, _NE), lambda i: (0, 0)),
            pl.BlockSpec((1, _NE), lambda i: (0, 0)),
        ],
        out_specs=pl.BlockSpec((tile, n_embed), lambda i: (i, 0)),
        out_shape=jax.ShapeDtypeStruct((n_tokens, n_embed), jnp.float32),
    )(x, proj, b_all, noise, bg.reshape(1, _NE), bn.reshape(1, _NE))
